# Initial kernel scaffold; baseline (speedup 1.0000x reference)
import jax
import jax.numpy as jnp
from jax.experimental import pallas as pl


def kernel(x, W):
    raise NotImplementedError("write your pallas kernel here")


# speedup vs baseline: 1.2128x; 1.2128x over previous
import functools

import jax
import jax.numpy as jnp
from jax.experimental import pallas as pl
from jax.experimental.pallas import tpu as pltpu

_D_MODEL = 4096
_NUM_EXPERTS = 64
_TOP_K = 8
_BLOCK_T = 1024


def _gate_kernel(x_ref, w_ref, topk_w_ref, probs_ref, topk_i_ref, logits_ref):
    logits = jax.lax.dot_general(
        x_ref[...], w_ref[...],
        dimension_numbers=(((1,), (1,)), ((), ())),
        preferred_element_type=jnp.float32,
    )
    logits_ref[...] = logits

    m = jnp.max(logits, axis=-1, keepdims=True)
    e = jnp.exp(logits - m)
    s = jnp.sum(e, axis=-1, keepdims=True)
    probs = e / s
    probs_ref[...] = probs
    psum = jnp.sum(probs, axis=-1, keepdims=True)

    eiota = jax.lax.broadcasted_iota(jnp.int32, probs.shape, 1)
    kiota = jax.lax.broadcasted_iota(jnp.int32, (probs.shape[0], _TOP_K), 1)
    out_w = jnp.zeros((probs.shape[0], _TOP_K), jnp.float32)
    out_i = jnp.zeros((probs.shape[0], _TOP_K), jnp.int32)
    cur = probs
    for k in range(_TOP_K):
        mx = jnp.max(cur, axis=-1, keepdims=True)
        idx = jnp.min(
            jnp.where(cur == mx, eiota, _NUM_EXPERTS), axis=-1, keepdims=True
        )
        out_w = jnp.where(kiota == k, mx, out_w)
        out_i = jnp.where(kiota == k, idx, out_i)
        if k + 1 < _TOP_K:
            cur = jnp.where(eiota == idx, -1.0, cur)
    topk_w_ref[...] = out_w / psum
    topk_i_ref[...] = out_i


@functools.partial(jax.jit, static_argnames=())
def kernel(x, W):
    n_tokens, d_model = x.shape
    n_experts = W.shape[0]
    grid = (n_tokens // _BLOCK_T,)
    out = pl.pallas_call(
        _gate_kernel,
        grid=grid,
        in_specs=[
            pl.BlockSpec((_BLOCK_T, d_model), lambda i: (i, 0)),
            pl.BlockSpec((n_experts, d_model), lambda i: (0, 0)),
        ],
        out_specs=[
            pl.BlockSpec((_BLOCK_T, _TOP_K), lambda i: (i, 0)),
            pl.BlockSpec((_BLOCK_T, n_experts), lambda i: (i, 0)),
            pl.BlockSpec((_BLOCK_T, _TOP_K), lambda i: (i, 0)),
            pl.BlockSpec((_BLOCK_T, n_experts), lambda i: (i, 0)),
        ],
        out_shape=[
            jax.ShapeDtypeStruct((n_tokens, _TOP_K), jnp.float32),
            jax.ShapeDtypeStruct((n_tokens, n_experts), jnp.float32),
            jax.ShapeDtypeStruct((n_tokens, _TOP_K), jnp.int32),
            jax.ShapeDtypeStruct((n_tokens, n_experts), jnp.float32),
        ],
        compiler_params=pltpu.CompilerParams(
            dimension_semantics=("arbitrary",),
        ),
    )(x, W)
    topk_w, probs, topk_i, logits = out
    return (topk_w, probs, topk_i, logits)


# speedup vs baseline: 1.2979x; 1.0702x over previous
import functools

import jax
import jax.numpy as jnp
from jax.experimental import pallas as pl
from jax.experimental.pallas import tpu as pltpu

_D_MODEL = 4096
_NUM_EXPERTS = 64
_TOP_K = 8
_BLOCK_T = 1024


def _gate_kernel(x_ref, w_ref, topk_w_ref, probs_ref, topk_i_ref, logits_ref):
    logits = jax.lax.dot_general(
        x_ref[...], w_ref[...],
        dimension_numbers=(((1,), (1,)), ((), ())),
        preferred_element_type=jnp.float32,
    )
    logits_ref[...] = logits

    m = jnp.max(logits, axis=-1, keepdims=True)
    e = jnp.exp(logits - m)
    s = jnp.sum(e, axis=-1, keepdims=True)
    probs = e / s
    probs_ref[...] = probs
    psum = jnp.sum(probs, axis=-1, keepdims=True)

    eiota = jax.lax.broadcasted_iota(jnp.int32, probs.shape, 1).astype(
        jnp.float32
    )
    kiota = jax.lax.broadcasted_iota(jnp.int32, (probs.shape[0], _TOP_K), 1)
    out_w = jnp.zeros((probs.shape[0], _TOP_K), jnp.float32)
    out_if = jnp.zeros((probs.shape[0], _TOP_K), jnp.float32)
    cur = probs
    for k in range(_TOP_K):
        mx = jnp.max(cur, axis=-1, keepdims=True)
        idxf = jnp.min(
            jnp.where(cur == mx, eiota, float(_NUM_EXPERTS)),
            axis=-1, keepdims=True,
        )
        out_w = jnp.where(kiota == k, mx, out_w)
        out_if = jnp.where(kiota == k, idxf, out_if)
        if k + 1 < _TOP_K:
            cur = jnp.where(eiota == idxf, -1.0, cur)
    topk_w_ref[...] = out_w / psum
    topk_i_ref[...] = out_if.astype(jnp.int32)


@functools.partial(jax.jit, static_argnames=())
def kernel(x, W):
    n_tokens, d_model = x.shape
    n_experts = W.shape[0]
    grid = (n_tokens // _BLOCK_T,)
    out = pl.pallas_call(
        _gate_kernel,
        grid=grid,
        in_specs=[
            pl.BlockSpec((_BLOCK_T, d_model), lambda i: (i, 0)),
            pl.BlockSpec((n_experts, d_model), lambda i: (0, 0)),
        ],
        out_specs=[
            pl.BlockSpec((_BLOCK_T, _TOP_K), lambda i: (i, 0)),
            pl.BlockSpec((_BLOCK_T, n_experts), lambda i: (i, 0)),
            pl.BlockSpec((_BLOCK_T, _TOP_K), lambda i: (i, 0)),
            pl.BlockSpec((_BLOCK_T, n_experts), lambda i: (i, 0)),
        ],
        out_shape=[
            jax.ShapeDtypeStruct((n_tokens, _TOP_K), jnp.float32),
            jax.ShapeDtypeStruct((n_tokens, n_experts), jnp.float32),
            jax.ShapeDtypeStruct((n_tokens, _TOP_K), jnp.int32),
            jax.ShapeDtypeStruct((n_tokens, n_experts), jnp.float32),
        ],
        compiler_params=pltpu.CompilerParams(
            dimension_semantics=("arbitrary",),
        ),
    )(x, W)
    topk_w, probs, topk_i, logits = out
    return (topk_w, probs, topk_i, logits)


# speedup vs baseline: 1.3602x; 1.0480x over previous
import functools

import jax
import jax.numpy as jnp
from jax.experimental import pallas as pl
from jax.experimental.pallas import tpu as pltpu

_D_MODEL = 4096
_NUM_EXPERTS = 64
_TOP_K = 8
_BLOCK_T = 1024


def _gate_kernel(x_ref, w_ref, topk_w_ref, probs_ref, topk_i_ref, logits_ref):
    logits_t = jax.lax.dot_general(
        w_ref[...], x_ref[...],
        dimension_numbers=(((1,), (1,)), ((), ())),
        preferred_element_type=jnp.float32,
    )
    logits_ref[...] = logits_t.T

    m = jnp.max(logits_t, axis=0, keepdims=True)
    e = jnp.exp(logits_t - m)
    s = jnp.sum(e, axis=0, keepdims=True)
    probs_t = e / s
    probs_ref[...] = probs_t.T
    psum = jnp.sum(probs_t, axis=0, keepdims=True)

    eiota = jax.lax.broadcasted_iota(jnp.int32, probs_t.shape, 0).astype(
        jnp.float32
    )
    cur = probs_t
    mxs = []
    idxs = []
    for k in range(_TOP_K):
        mx = jnp.max(cur, axis=0, keepdims=True)
        idxf = jnp.min(
            jnp.where(cur == mx, eiota, float(_NUM_EXPERTS)),
            axis=0, keepdims=True,
        )
        mxs.append(mx / psum)
        idxs.append(idxf)
        if k + 1 < _TOP_K:
            cur = jnp.where(eiota == idxf, -1.0, cur)
    topk_w_t = jnp.concatenate(mxs, axis=0)
    topk_i_t = jnp.concatenate(idxs, axis=0)
    topk_w_ref[...] = topk_w_t.T
    topk_i_ref[...] = topk_i_t.T.astype(jnp.int32)


@functools.partial(jax.jit, static_argnames=())
def kernel(x, W):
    n_tokens, d_model = x.shape
    n_experts = W.shape[0]
    grid = (n_tokens // _BLOCK_T,)
    out = pl.pallas_call(
        _gate_kernel,
        grid=grid,
        in_specs=[
            pl.BlockSpec((_BLOCK_T, d_model), lambda i: (i, 0)),
            pl.BlockSpec((n_experts, d_model), lambda i: (0, 0)),
        ],
        out_specs=[
            pl.BlockSpec((_BLOCK_T, _TOP_K), lambda i: (i, 0)),
            pl.BlockSpec((_BLOCK_T, n_experts), lambda i: (i, 0)),
            pl.BlockSpec((_BLOCK_T, _TOP_K), lambda i: (i, 0)),
            pl.BlockSpec((_BLOCK_T, n_experts), lambda i: (i, 0)),
        ],
        out_shape=[
            jax.ShapeDtypeStruct((n_tokens, _TOP_K), jnp.float32),
            jax.ShapeDtypeStruct((n_tokens, n_experts), jnp.float32),
            jax.ShapeDtypeStruct((n_tokens, _TOP_K), jnp.int32),
            jax.ShapeDtypeStruct((n_tokens, n_experts), jnp.float32),
        ],
        compiler_params=pltpu.CompilerParams(
            dimension_semantics=("arbitrary",),
        ),
    )(x, W)
    topk_w, probs, topk_i, logits = out
    return (topk_w, probs, topk_i, logits)
